# baseline (speedup 1.0000x reference)
import functools
import math

import jax
import jax.numpy as jnp
from jax import lax
from jax.experimental import pallas as pl
from jax.experimental.pallas import tpu as pltpu
from jax.experimental.pallas import tpu_sc as plsc

VOCAB = 1000000
D_MODEL = 64
ROWS = 4096
COLS = 200
B = ROWS * COLS
NC = 2
NS = 16
NW = NC * NS
XPW = ROWS // NW
BPW = B // NW
CHUNK = COLS
NCH = XPW
NBUF = 4
SCALE = math.sqrt(D_MODEL)

_C0S = tuple(range(0, 192, 16)) + (184,)

_mesh = plsc.VectorSubcoreMesh(core_axis_name="c", subcore_axis_name="s")


@functools.partial(
    pl.kernel,
    mesh=_mesh,
    out_type=jax.ShapeDtypeStruct((ROWS, D_MODEL, COLS), jnp.float32),
    scratch_types=[pltpu.VMEM((BPW,), jnp.int32)]
    + [pltpu.VMEM((CHUNK, D_MODEL), jnp.float32)] * NBUF
    + [pltpu.VMEM((D_MODEL, CHUNK), jnp.float32)] * NBUF
    + [pltpu.SemaphoreType.DMA] * (2 * NBUF),
    compiler_params=pltpu.CompilerParams(use_tc_tiling_on_sc=False,
                                         needs_layout_passes=False),
)
def _embed(x_hbm, lut_hbm, out_hbm, idx_v,
           r0, r1, r2, r3, t0, t1, t2, t3,
           g0, g1, g2, g3, s0, s1, s2, s3):
    rows = (r0, r1, r2, r3)
    tbufs = (t0, t1, t2, t3)
    gsem = (g0, g1, g2, g3)
    ssem = (s0, s1, s2, s3)
    wid = lax.axis_index("s") * NC + lax.axis_index("c")
    xbase = wid * XPW
    pltpu.sync_copy(x_hbm.at[pl.ds(wid * BPW, BPW)], idx_v)
    lane = lax.iota(jnp.int32, 16)

    def start_gather(g, b):
        pltpu.async_copy(
            lut_hbm.at[idx_v.at[pl.ds(g * CHUNK, CHUNK)]], rows[b], gsem[b])

    def wait_gather(b):
        pltpu.make_async_copy(
            lut_hbm.at[idx_v.at[pl.ds(0, CHUNK)]], rows[b], gsem[b]).wait()

    def wait_store(b):
        pltpu.make_async_copy(tbufs[b], out_hbm.at[xbase], ssem[b]).wait()

    lanes_c0 = tuple(lane + c0 for c0 in _C0S)

    def transpose_scale(src, dst):
        @plsc.parallel_loop(0, D_MODEL, step=1, unroll=4)
        def body(d):
            dvec = jnp.full((16,), 0, jnp.int32) + d
            vs = [plsc.load_gather(src, [lanes_c0[i], dvec])
                  for i in range(len(_C0S))]
            for i, c0 in enumerate(_C0S):
                dst[d, pl.ds(c0, 16)] = vs[i] * SCALE

    for b in range(NBUF - 1):
        start_gather(b, b)

    def group(t, carry):
        for bb in range(NBUF):
            g = t * NBUF + bb
            gl = g + NBUF - 1

            @pl.when(gl < NCH)
            def _():
                start_gather(gl, (bb + NBUF - 1) % NBUF)

            wait_gather(bb)

            @pl.when(g >= NBUF)
            def _():
                wait_store(bb)

            transpose_scale(rows[bb], tbufs[bb])
            pltpu.async_copy(tbufs[bb], out_hbm.at[xbase + g], ssem[bb])
        return carry

    lax.fori_loop(0, NCH // NBUF, group, 0)
    for b in range(NBUF):
        wait_store(b)


def kernel(x, lut):
    out_t = _embed(x.reshape(B), lut)
    return out_t.transpose(0, 2, 1)


# speedup vs baseline: 2.2962x; 2.2962x over previous
import functools
import math

import jax
import jax.numpy as jnp
from jax import lax
from jax.experimental import pallas as pl
from jax.experimental.pallas import tpu as pltpu
from jax.experimental.pallas import tpu_sc as plsc

VOCAB = 1000000
D_MODEL = 64
DPAD = 128
ROWS = 4096
COLS = 200
B = ROWS * COLS
NC = 2
NS = 16
NW = NC * NS
XPW = ROWS // NW
BPW = B // NW
CHUNK = COLS
NCH = XPW
NBUF = 4
SCALE = math.sqrt(D_MODEL)

_mesh = plsc.VectorSubcoreMesh(core_axis_name="c", subcore_axis_name="s")


@functools.partial(
    pl.kernel,
    mesh=_mesh,
    out_type=jax.ShapeDtypeStruct((ROWS, COLS, DPAD), jnp.float32),
    scratch_types=[pltpu.VMEM((BPW,), jnp.int32)]
    + [pltpu.VMEM((CHUNK, D_MODEL), jnp.float32)] * NBUF
    + [pltpu.SemaphoreType.DMA] * (2 * NBUF),
    compiler_params=pltpu.CompilerParams(use_tc_tiling_on_sc=False,
                                         needs_layout_passes=False),
)
def _embed(x_hbm, lut_hbm, out_hbm, idx_v,
           r0, r1, r2, r3, g0, g1, g2, g3, s0, s1, s2, s3):
    rows = (r0, r1, r2, r3)
    gsem = (g0, g1, g2, g3)
    ssem = (s0, s1, s2, s3)
    wid = lax.axis_index("s") * NC + lax.axis_index("c")
    xbase = wid * XPW
    pltpu.sync_copy(x_hbm.at[pl.ds(wid * BPW, BPW)], idx_v)

    def start_gather(g, b):
        pltpu.async_copy(
            lut_hbm.at[idx_v.at[pl.ds(g * CHUNK, CHUNK)]], rows[b], gsem[b])

    def wait_gather(b):
        pltpu.make_async_copy(
            lut_hbm.at[idx_v.at[pl.ds(0, CHUNK)]], rows[b], gsem[b]).wait()

    def out_block(g):
        return out_hbm.at[xbase + g, :, pl.ds(0, D_MODEL)]

    def wait_store(b):
        pltpu.make_async_copy(rows[b], out_block(0), ssem[b]).wait()

    def scale(buf):
        def body(i, c):
            r = i * 4
            for k in range(4):
                for j in range(D_MODEL // 16):
                    sl = pl.ds(j * 16, 16)
                    buf[r + k, sl] = buf[r + k, sl] * SCALE
            return c
        lax.fori_loop(0, CHUNK // 4, body, 0)

    for b in range(NBUF - 1):
        start_gather(b, b)

    def group(t, carry):
        for bb in range(NBUF):
            g = t * NBUF + bb
            gl = g + NBUF - 1
            bl = (bb + NBUF - 1) % NBUF

            @pl.when(gl < NCH)
            def _():
                @pl.when(gl >= NBUF)
                def _():
                    wait_store(bl)
                start_gather(gl, bl)

            wait_gather(bb)
            scale(rows[bb])
            pltpu.async_copy(rows[bb], out_block(g), ssem[bb])
        return carry

    lax.fori_loop(0, NCH // NBUF, group, 0)
    for b in range(NBUF):
        wait_store(b)


def kernel(x, lut):
    out = _embed(x.reshape(B), lut)
    return out[:, :, :D_MODEL]
